# Initial kernel scaffold; baseline (speedup 1.0000x reference)
import jax
import jax.numpy as jnp
from jax.experimental import pallas as pl


def kernel(user_emb, item_emb, Wg1, bg1, Wg2, bg2, Wb_buy, bb_buy, Wb_view, bb_view, uH_buy, iH_buy, uH_view, iH_view, all_edge_index, edge_index_buy, edge_index_view):
    raise NotImplementedError("write your pallas kernel here")


# speedup vs baseline: 6.0125x; 6.0125x over previous
import functools

import jax
import jax.numpy as jnp
from jax import lax
from jax.experimental import pallas as pl
from jax.experimental.pallas import tpu as pltpu
from jax.experimental.pallas import tpu_sc as plsc

N_USERS = 4999
N_ITEMS = 4999
D = 128
N = N_USERS + 1 + N_ITEMS + 1
NU = N_USERS + 1

NC = 2
NS = 16
NW = NC * NS
CH = 128

NPAD = 10240
HALF = NPAD // 2
HS = HALF + 128
ZCH = HALF // NS

_MESH = plsc.VectorSubcoreMesh(core_axis_name="c", subcore_axis_name="s")


CPW = 40


def _shard_edges(src, dst):
    e = src.shape[0]
    k = -(-e // (NW * CPW * CH))
    te = k * NW * CPW * CH
    pad = te - e
    ar = jnp.arange(pad, dtype=jnp.int32)
    src_p = jnp.concatenate([src.astype(jnp.int32), ar % 9973])
    dst_p = jnp.concatenate([dst.astype(jnp.int32), (NPAD - 16) + (ar % 16)])
    src_p = src_p.reshape(k, NW, CPW, CH)
    dst_p = dst_p.reshape(k, NW, CPW, CH)
    return [(src_p[i], dst_p[i]) for i in range(k)]


def _sc_agg(table, src_sh, dst_sh, zeros_hbm):
    @functools.partial(
        pl.kernel,
        mesh=_MESH,
        out_type=jax.ShapeDtypeStruct((NC * NPAD, D), jnp.float32),
        scratch_types=[
            pltpu.VMEM((CPW, CH), jnp.int32),
            pltpu.VMEM((CPW, CH), jnp.int32),
            pltpu.VMEM((CPW, CH), jnp.int32),
            pltpu.VMEM((CH, D), jnp.float32),
            pltpu.VMEM((ZCH, D), jnp.float32),
            pltpu.VMEM_SHARED((HS, D), jnp.float32),
            pltpu.SemaphoreType.DMA,
        ],
    )
    def k(tab_h, src_h, dst_h, zer_h, out_h, src_v, dst_v, dst2_v, rows_v,
          stage_v, acc_sh, sem):
        c = lax.axis_index("c")
        s = lax.axis_index("s")
        wid = s * NC + c
        pltpu.sync_copy(src_h.at[wid], src_v)
        pltpu.sync_copy(dst_h.at[wid], dst_v)

        for h in (0, 1):
            base = h * HALF
            pltpu.sync_copy(zer_h, stage_v)
            pltpu.sync_copy(stage_v, acc_sh.at[pl.ds(s * ZCH, ZCH)])
            plsc.subcore_barrier()

            def body(j, carry):
                for kk in range(CH // 16):
                    v = dst_v[j, pl.ds(kk * 16, 16)]
                    local = v - base
                    ok = (local >= 0) & (local < HALF)
                    idx = jnp.where(ok, local, HALF + (v & 15))
                    dst2_v[j, pl.ds(kk * 16, 16)] = idx
                pltpu.async_copy(tab_h.at[src_v.at[j]], rows_v, sem).wait()
                pltpu.sync_copy(rows_v, acc_sh.at[dst2_v.at[j]], add=True)
                return carry

            lax.fori_loop(0, CPW, body, 0)
            plsc.subcore_barrier()
            pltpu.sync_copy(acc_sh.at[pl.ds(s * ZCH, ZCH)], stage_v)
            pltpu.sync_copy(
                stage_v,
                out_h.at[pl.ds(c * NPAD + base + s * ZCH, ZCH)],
            )
            plsc.subcore_barrier()

    return k(table, src_sh, dst_sh, zeros_hbm)


def _dis_body(p_ref, o_ref):
    p = p_ref[...]
    deg = jnp.stack([
        jnp.sum(p[0:4], axis=0),
        jnp.sum(p[4:6], axis=0),
        jnp.sum(p[6:8], axis=0),
    ])
    o_ref[...] = jnp.where(
        deg > 0, 1.0 / jnp.sqrt(jnp.maximum(deg, 1e-12)), 0.0
    )


def _tc_dis(degp):
    return pl.pallas_call(
        _dis_body,
        out_shape=jax.ShapeDtypeStruct((3, NPAD), jnp.float32),
    )(degp)


def _scale_mm_body(x_ref, w_ref, s_ref, o_ref):
    o_ref[...] = s_ref[...] * jnp.dot(
        x_ref[...], w_ref[...], preferred_element_type=jnp.float32
    )


def _tc_scale_mm(x, w, s):
    return pl.pallas_call(
        _scale_mm_body,
        out_shape=jax.ShapeDtypeStruct(x.shape, jnp.float32),
    )(x, w, s)


def _post_body(scale, p_ref, s_ref, b_ref, base_ref, y_ref, acc_ref):
    t = s_ref[...] * jnp.sum(p_ref[...], axis=0) + b_ref[...]
    nrm = jnp.sqrt(jnp.sum(t * t, axis=-1, keepdims=True))
    y = t / jnp.maximum(nrm, 1e-12)
    y_ref[...] = y
    acc_ref[...] = base_ref[...] + scale * y


def _tc_post(p, dis, b, base, scale):
    return pl.pallas_call(
        functools.partial(_post_body, scale),
        out_shape=[
            jax.ShapeDtypeStruct((N, D), jnp.float32),
            jax.ShapeDtypeStruct((N, D), jnp.float32),
        ],
    )(p, dis, b.reshape(1, D), base)


def _hyper_body(g_ref, uh_ref, ih_ref, o_ref):
    u = g_ref[0:NU, :]
    it = g_ref[NU:N, :]
    hu = jnp.dot(u, uh_ref[...], preferred_element_type=jnp.float32)
    hi = jnp.dot(it, ih_ref[...], preferred_element_type=jnp.float32)
    au = lax.dot_general(hu, u, (((0,), (0,)), ((), ())),
                         preferred_element_type=jnp.float32)
    ai = lax.dot_general(hi, it, (((0,), (0,)), ((), ())),
                         preferred_element_type=jnp.float32)
    o_ref[0:NU, :] = jnp.dot(hu, au, preferred_element_type=jnp.float32)
    o_ref[NU:N, :] = jnp.dot(hi, ai, preferred_element_type=jnp.float32)


def _tc_hyper(gcn, uh, ih):
    return pl.pallas_call(
        _hyper_body,
        out_shape=jax.ShapeDtypeStruct((N, D), jnp.float32),
    )(gcn, uh, ih)


def _cascade_body(temp_ref, col_ref, sem_ref, o_ref):
    col = col_ref[...]
    sem = sem_ref[...]
    num = jnp.sum(sem * col, axis=-1, keepdims=True)
    den = jnp.sum(col * col, axis=-1, keepdims=True) + 1e-08
    o_ref[...] = temp_ref[...] + col + (num / den) * col


def _tc_cascade(temp, gcn, sem):
    return pl.pallas_call(
        _cascade_body,
        out_shape=jax.ShapeDtypeStruct((N, D), jnp.float32),
    )(temp, gcn, sem)


def kernel(user_emb, item_emb, Wg1, bg1, Wg2, bg2, Wb_buy, bb_buy, Wb_view,
           bb_view, uH_buy, iH_buy, uH_view, iH_view, all_edge_index,
           edge_index_buy, edge_index_view):
    x0 = jnp.concatenate([user_emb, item_emb], axis=0)

    ei_g = all_edge_index.astype(jnp.int32)
    sh_g = _shard_edges(ei_g[0], ei_g[1])
    sh_b = _shard_edges(edge_index_buy[0], edge_index_buy[1])
    sh_v = _shard_edges(edge_index_view[0], edge_index_view[1])

    ones_t = jnp.ones((N, D), jnp.float32)
    zer = jnp.zeros((ZCH, D), jnp.float32)

    def agg_graph(table, shards):
        def one(s_, d_):
            nonlocal zer
            p = _sc_agg(table, s_, d_, zer).reshape(NC, NPAD, D)
            zer = lax.optimization_barrier((zer, p[0, :ZCH]))[0]
            return p

        return jnp.concatenate([one(s_, d_) for (s_, d_) in shards], axis=0)

    degp = jnp.concatenate([
        agg_graph(ones_t, sh_g)[:, :, 0],
        agg_graph(ones_t, sh_b)[:, :, 0],
        agg_graph(ones_t, sh_v)[:, :, 0],
    ], axis=0)
    dis_all = _tc_dis(degp)
    dis_g = dis_all[0, :N][:, None]
    dis_buy = dis_all[1, :N][:, None]
    dis_view = dis_all[2, :N][:, None]

    def conv(x, w, b, dis, shards, base, scale):
        h = _tc_scale_mm(x, w, dis)
        p = agg_graph(h, shards)[:, :N, :]
        return _tc_post(p, dis, b, base, scale)

    x1, acc1 = conv(x0, Wg1, bg1, dis_g, sh_g, x0, 1.0)
    _, g = conv(x1, Wg2, bg2, dis_g, sh_g, acc1, 0.5)

    cascading = g
    outs = []
    for (w, b, uh, ih, dis, shards) in (
        (Wb_buy, bb_buy, uH_buy, iH_buy, dis_buy, sh_b),
        (Wb_view, bb_view, uH_view, iH_view, dis_view, sh_v),
    ):
        _, gcn_emb = conv(cascading, w, b, dis, shards, cascading, 1.0)
        sem = _tc_hyper(gcn_emb, uh, ih)
        cascading = _tc_cascade(cascading, gcn_emb, sem)
        outs.append(cascading)

    final_user = jnp.stack([o[:NU] for o in outs], axis=1)
    final_item = jnp.stack([o[NU:] for o in outs], axis=1)
    return (final_user, final_item)


# speedup vs baseline: 9.9463x; 1.6543x over previous
import functools

import jax
import jax.numpy as jnp
from jax import lax
from jax.experimental import pallas as pl
from jax.experimental.pallas import tpu as pltpu
from jax.experimental.pallas import tpu_sc as plsc

N_USERS = 4999
N_ITEMS = 4999
D = 128
N = N_USERS + 1 + N_ITEMS + 1
NU = N_USERS + 1

NC = 2
NS = 16
NW = NC * NS
CH = 128

NPAD = 10240
HALF = NPAD // 2
HS = HALF + 128
ZCH = HALF // NS
ZS = ZCH // 2
NBUF = 3

_MESH = plsc.VectorSubcoreMesh(core_axis_name="c", subcore_axis_name="s")


CPW = 40


def _shard_edges(src, dst):
    e = src.shape[0]
    k = -(-e // (NW * CPW * CH))
    te = k * NW * CPW * CH
    pad = te - e
    ar = jnp.arange(pad, dtype=jnp.int32)
    src_p = jnp.concatenate([src.astype(jnp.int32), ar % 9973])
    dst_p = jnp.concatenate([dst.astype(jnp.int32), (NPAD - 16) + (ar % 16)])
    src_p = src_p.reshape(k, NW, CPW, CH)
    dst_p = dst_p.reshape(k, NW, CPW, CH)
    return [(src_p[i], dst_p[i]) for i in range(k)]


def _sc_agg(table, src_sh, dst_sh, zeros_hbm):
    @functools.partial(
        pl.kernel,
        mesh=_MESH,
        out_type=jax.ShapeDtypeStruct((NC * NPAD, D), jnp.float32),
        scratch_types=[
            pltpu.VMEM((CPW, CH), jnp.int32),
            pltpu.VMEM((CPW, CH), jnp.int32),
            pltpu.VMEM((CPW, CH), jnp.int32),
            pltpu.VMEM((NBUF, CH, D), jnp.float32),
            pltpu.VMEM((ZS, D), jnp.float32),
            pltpu.VMEM_SHARED((HS, D), jnp.float32),
            pltpu.SemaphoreType.DMA,
        ],
    )
    def k(tab_h, src_h, dst_h, zer_h, out_h, src_v, dst_v, dst2_v, rows_v,
          stage_v, acc_sh, sem):
        c = lax.axis_index("c")
        s = lax.axis_index("s")
        wid = s * NC + c
        pltpu.sync_copy(src_h.at[wid], src_v)
        pltpu.sync_copy(dst_h.at[wid], dst_v)

        def fire(jj):
            pltpu.async_copy(
                tab_h.at[src_v.at[jj]], rows_v.at[lax.rem(jj, NBUF)], sem
            )

        for h in (0, 1):
            base = h * HALF
            pltpu.sync_copy(zer_h, stage_v)
            pltpu.sync_copy(stage_v, acc_sh.at[pl.ds(s * ZCH, ZS)])
            pltpu.sync_copy(stage_v, acc_sh.at[pl.ds(s * ZCH + ZS, ZS)])
            plsc.subcore_barrier()
            fire(0)
            fire(1)

            def body(j, carry):
                @pl.when(j < CPW - 2)
                def _():
                    fire(j + 2)

                for kk in range(CH // 16):
                    v = dst_v[j, pl.ds(kk * 16, 16)]
                    local = v - base
                    ok = (local >= 0) & (local < HALF)
                    idx = jnp.where(ok, local, HALF + (v & 15))
                    dst2_v[j, pl.ds(kk * 16, 16)] = idx
                buf = rows_v.at[lax.rem(j, NBUF)]
                pltpu.make_async_copy(tab_h.at[pl.ds(0, CH)], buf, sem).wait()
                pltpu.sync_copy(buf, acc_sh.at[dst2_v.at[j]], add=True)
                return carry

            lax.fori_loop(0, CPW, body, 0)
            plsc.subcore_barrier()
            for hh in (0, 1):
                pltpu.sync_copy(
                    acc_sh.at[pl.ds(s * ZCH + hh * ZS, ZS)], stage_v
                )
                pltpu.sync_copy(
                    stage_v,
                    out_h.at[pl.ds(c * NPAD + base + s * ZCH + hh * ZS, ZS)],
                )
            plsc.subcore_barrier()

    return k(table, src_sh, dst_sh, zeros_hbm)


def _dis_body(p_ref, o_ref):
    p = p_ref[...]
    deg = jnp.stack([
        jnp.sum(p[0:4], axis=0),
        jnp.sum(p[4:6], axis=0),
        jnp.sum(p[6:8], axis=0),
    ])
    o_ref[...] = jnp.where(
        deg > 0, 1.0 / jnp.sqrt(jnp.maximum(deg, 1e-12)), 0.0
    )


def _tc_dis(degp):
    return pl.pallas_call(
        _dis_body,
        out_shape=jax.ShapeDtypeStruct((3, NPAD), jnp.float32),
    )(degp)


def _scale_mm_body(x_ref, w_ref, s_ref, o_ref):
    o_ref[...] = s_ref[...] * jnp.dot(
        x_ref[...], w_ref[...], preferred_element_type=jnp.float32
    )


def _tc_scale_mm(x, w, s):
    return pl.pallas_call(
        _scale_mm_body,
        out_shape=jax.ShapeDtypeStruct(x.shape, jnp.float32),
    )(x, w, s)


def _post_body(scale, p_ref, s_ref, b_ref, base_ref, y_ref, acc_ref):
    t = s_ref[...] * jnp.sum(p_ref[...], axis=0) + b_ref[...]
    nrm = jnp.sqrt(jnp.sum(t * t, axis=-1, keepdims=True))
    y = t / jnp.maximum(nrm, 1e-12)
    y_ref[...] = y
    acc_ref[...] = base_ref[...] + scale * y


def _tc_post(p, dis, b, base, scale):
    return pl.pallas_call(
        functools.partial(_post_body, scale),
        out_shape=[
            jax.ShapeDtypeStruct((N, D), jnp.float32),
            jax.ShapeDtypeStruct((N, D), jnp.float32),
        ],
    )(p, dis, b.reshape(1, D), base)


def _hyper_body(g_ref, uh_ref, ih_ref, o_ref):
    u = g_ref[0:NU, :]
    it = g_ref[NU:N, :]
    hu = jnp.dot(u, uh_ref[...], preferred_element_type=jnp.float32)
    hi = jnp.dot(it, ih_ref[...], preferred_element_type=jnp.float32)
    au = lax.dot_general(hu, u, (((0,), (0,)), ((), ())),
                         preferred_element_type=jnp.float32)
    ai = lax.dot_general(hi, it, (((0,), (0,)), ((), ())),
                         preferred_element_type=jnp.float32)
    o_ref[0:NU, :] = jnp.dot(hu, au, preferred_element_type=jnp.float32)
    o_ref[NU:N, :] = jnp.dot(hi, ai, preferred_element_type=jnp.float32)


def _tc_hyper(gcn, uh, ih):
    return pl.pallas_call(
        _hyper_body,
        out_shape=jax.ShapeDtypeStruct((N, D), jnp.float32),
    )(gcn, uh, ih)


def _cascade_body(temp_ref, col_ref, sem_ref, o_ref):
    col = col_ref[...]
    sem = sem_ref[...]
    num = jnp.sum(sem * col, axis=-1, keepdims=True)
    den = jnp.sum(col * col, axis=-1, keepdims=True) + 1e-08
    o_ref[...] = temp_ref[...] + col + (num / den) * col


def _tc_cascade(temp, gcn, sem):
    return pl.pallas_call(
        _cascade_body,
        out_shape=jax.ShapeDtypeStruct((N, D), jnp.float32),
    )(temp, gcn, sem)


def kernel(user_emb, item_emb, Wg1, bg1, Wg2, bg2, Wb_buy, bb_buy, Wb_view,
           bb_view, uH_buy, iH_buy, uH_view, iH_view, all_edge_index,
           edge_index_buy, edge_index_view):
    x0 = jnp.concatenate([user_emb, item_emb], axis=0)

    ei_g = all_edge_index.astype(jnp.int32)
    sh_g = _shard_edges(ei_g[0], ei_g[1])
    sh_b = _shard_edges(edge_index_buy[0], edge_index_buy[1])
    sh_v = _shard_edges(edge_index_view[0], edge_index_view[1])

    ones_t = jnp.ones((N, D), jnp.float32)
    zer = jnp.zeros((ZS, D), jnp.float32)

    def agg_graph(table, shards):
        def one(s_, d_):
            nonlocal zer
            p = _sc_agg(table, s_, d_, zer).reshape(NC, NPAD, D)
            zer = lax.optimization_barrier((zer, p[0, :ZS]))[0]
            return p

        return jnp.concatenate([one(s_, d_) for (s_, d_) in shards], axis=0)

    degp = jnp.concatenate([
        agg_graph(ones_t, sh_g)[:, :, 0],
        agg_graph(ones_t, sh_b)[:, :, 0],
        agg_graph(ones_t, sh_v)[:, :, 0],
    ], axis=0)
    dis_all = _tc_dis(degp)
    dis_g = dis_all[0, :N][:, None]
    dis_buy = dis_all[1, :N][:, None]
    dis_view = dis_all[2, :N][:, None]

    def conv(x, w, b, dis, shards, base, scale):
        h = _tc_scale_mm(x, w, dis)
        p = agg_graph(h, shards)[:, :N, :]
        return _tc_post(p, dis, b, base, scale)

    x1, acc1 = conv(x0, Wg1, bg1, dis_g, sh_g, x0, 1.0)
    _, g = conv(x1, Wg2, bg2, dis_g, sh_g, acc1, 0.5)

    cascading = g
    outs = []
    for (w, b, uh, ih, dis, shards) in (
        (Wb_buy, bb_buy, uH_buy, iH_buy, dis_buy, sh_b),
        (Wb_view, bb_view, uH_view, iH_view, dis_view, sh_v),
    ):
        _, gcn_emb = conv(cascading, w, b, dis, shards, cascading, 1.0)
        sem = _tc_hyper(gcn_emb, uh, ih)
        cascading = _tc_cascade(cascading, gcn_emb, sem)
        outs.append(cascading)

    final_user = jnp.stack([o[:NU] for o in outs], axis=1)
    final_item = jnp.stack([o[NU:] for o in outs], axis=1)
    return (final_user, final_item)


# speedup vs baseline: 9.9549x; 1.0009x over previous
import functools

import jax
import jax.numpy as jnp
from jax import lax
from jax.experimental import pallas as pl
from jax.experimental.pallas import tpu as pltpu
from jax.experimental.pallas import tpu_sc as plsc

N_USERS = 4999
N_ITEMS = 4999
D = 128
N = N_USERS + 1 + N_ITEMS + 1
NU = N_USERS + 1

NC = 2
NS = 16
NW = NC * NS
CH = 128

NPAD = 10240
HALF = NPAD // 2
HS = HALF + 128
ZCH = HALF // NS
ZS = ZCH // 2
NBUF = 3

_MESH = plsc.VectorSubcoreMesh(core_axis_name="c", subcore_axis_name="s")


CPW = 40


def _shard_edges(src, dst):
    e = src.shape[0]
    k = -(-e // (NW * CPW * CH))
    te = k * NW * CPW * CH
    pad = te - e
    ar = jnp.arange(pad, dtype=jnp.int32)
    src_p = jnp.concatenate([src.astype(jnp.int32), ar % 9973])
    dst_p = jnp.concatenate([dst.astype(jnp.int32), NPAD + (ar % 16)])
    src_p = src_p.reshape(k, NW, CPW, CH)
    dst_p = dst_p.reshape(k, NW, CPW, CH)
    return [(src_p[i], dst_p[i]) for i in range(k)]


def _sc_agg(table, src_sh, dst_sh, zeros_hbm):
    @functools.partial(
        pl.kernel,
        mesh=_MESH,
        out_type=jax.ShapeDtypeStruct((NC * NPAD, D), jnp.float32),
        scratch_types=[
            pltpu.VMEM((CPW, CH), jnp.int32),
            pltpu.VMEM((CPW, CH), jnp.int32),
            pltpu.VMEM((CPW, CH), jnp.int32),
            pltpu.VMEM((NBUF, CH, D), jnp.float32),
            pltpu.VMEM((ZS, D), jnp.float32),
            pltpu.VMEM_SHARED((HS, D), jnp.float32),
            pltpu.SemaphoreType.DMA,
        ],
    )
    def k(tab_h, src_h, dst_h, zer_h, out_h, src_v, dst_v, dst2_v, rows_v,
          stage_v, acc_sh, sem):
        c = lax.axis_index("c")
        s = lax.axis_index("s")
        wid = s * NC + c
        pltpu.sync_copy(src_h.at[wid], src_v)
        pltpu.sync_copy(dst_h.at[wid], dst_v)

        def fire(jj):
            pltpu.async_copy(
                tab_h.at[src_v.at[jj]], rows_v.at[lax.rem(jj, NBUF)], sem
            )

        for h in (0, 1):
            base = h * HALF
            pltpu.sync_copy(zer_h, stage_v)
            pltpu.sync_copy(stage_v, acc_sh.at[pl.ds(s * ZCH, ZS)])
            pltpu.sync_copy(stage_v, acc_sh.at[pl.ds(s * ZCH + ZS, ZS)])
            plsc.subcore_barrier()
            fire(0)
            fire(1)

            def body(j, carry):
                @pl.when(j < CPW - 2)
                def _():
                    fire(j + 2)

                for kk in range(CH // 16):
                    v = dst_v[j, pl.ds(kk * 16, 16)]
                    local = v - base
                    ok = (local >= 0) & (local < HALF)
                    idx = jnp.where(ok, local, HALF + (v & 15))
                    dst2_v[j, pl.ds(kk * 16, 16)] = idx
                buf = rows_v.at[lax.rem(j, NBUF)]
                pltpu.make_async_copy(tab_h.at[pl.ds(0, CH)], buf, sem).wait()
                pltpu.sync_copy(buf, acc_sh.at[dst2_v.at[j]], add=True)
                return carry

            lax.fori_loop(0, CPW, body, 0)
            plsc.subcore_barrier()
            for hh in (0, 1):
                pltpu.sync_copy(
                    acc_sh.at[pl.ds(s * ZCH + hh * ZS, ZS)], stage_v
                )
                pltpu.sync_copy(
                    stage_v,
                    out_h.at[pl.ds(c * NPAD + base + s * ZCH + hh * ZS, ZS)],
                )
            plsc.subcore_barrier()

    return k(table, src_sh, dst_sh, zeros_hbm)


def _dis_body(p_ref, o_ref):
    p = p_ref[...]
    deg = jnp.stack([
        jnp.sum(p[0:4], axis=0),
        jnp.sum(p[4:6], axis=0),
        jnp.sum(p[6:8], axis=0),
    ])
    o_ref[...] = jnp.where(
        deg > 0, 1.0 / jnp.sqrt(jnp.maximum(deg, 1e-12)), 0.0
    )


def _tc_dis(degp):
    return pl.pallas_call(
        _dis_body,
        out_shape=jax.ShapeDtypeStruct((3, NPAD), jnp.float32),
    )(degp)


def _scale_mm_body(x_ref, w_ref, s_ref, o_ref):
    o_ref[...] = s_ref[...] * jnp.dot(
        x_ref[...], w_ref[...], preferred_element_type=jnp.float32
    )


def _tc_scale_mm(x, w, s):
    return pl.pallas_call(
        _scale_mm_body,
        out_shape=jax.ShapeDtypeStruct(x.shape, jnp.float32),
    )(x, w, s)


def _post_body(scale, p_ref, s_ref, b_ref, base_ref, y_ref, acc_ref):
    t = s_ref[...] * jnp.sum(p_ref[...], axis=0) + b_ref[...]
    nrm = jnp.sqrt(jnp.sum(t * t, axis=-1, keepdims=True))
    y = t / jnp.maximum(nrm, 1e-12)
    y_ref[...] = y
    acc_ref[...] = base_ref[...] + scale * y


def _tc_post(p, dis, b, base, scale):
    return pl.pallas_call(
        functools.partial(_post_body, scale),
        out_shape=[
            jax.ShapeDtypeStruct((N, D), jnp.float32),
            jax.ShapeDtypeStruct((N, D), jnp.float32),
        ],
    )(p, dis, b.reshape(1, D), base)


def _hyper_body(g_ref, uh_ref, ih_ref, o_ref):
    u = g_ref[0:NU, :]
    it = g_ref[NU:N, :]
    hu = jnp.dot(u, uh_ref[...], preferred_element_type=jnp.float32)
    hi = jnp.dot(it, ih_ref[...], preferred_element_type=jnp.float32)
    au = lax.dot_general(hu, u, (((0,), (0,)), ((), ())),
                         preferred_element_type=jnp.float32)
    ai = lax.dot_general(hi, it, (((0,), (0,)), ((), ())),
                         preferred_element_type=jnp.float32)
    o_ref[0:NU, :] = jnp.dot(hu, au, preferred_element_type=jnp.float32)
    o_ref[NU:N, :] = jnp.dot(hi, ai, preferred_element_type=jnp.float32)


def _tc_hyper(gcn, uh, ih):
    return pl.pallas_call(
        _hyper_body,
        out_shape=jax.ShapeDtypeStruct((N, D), jnp.float32),
    )(gcn, uh, ih)


def _cascade_body(temp_ref, col_ref, sem_ref, o_ref):
    col = col_ref[...]
    sem = sem_ref[...]
    num = jnp.sum(sem * col, axis=-1, keepdims=True)
    den = jnp.sum(col * col, axis=-1, keepdims=True) + 1e-08
    o_ref[...] = temp_ref[...] + col + (num / den) * col


def _tc_cascade(temp, gcn, sem):
    return pl.pallas_call(
        _cascade_body,
        out_shape=jax.ShapeDtypeStruct((N, D), jnp.float32),
    )(temp, gcn, sem)


def kernel(user_emb, item_emb, Wg1, bg1, Wg2, bg2, Wb_buy, bb_buy, Wb_view,
           bb_view, uH_buy, iH_buy, uH_view, iH_view, all_edge_index,
           edge_index_buy, edge_index_view):
    x0 = jnp.concatenate([user_emb, item_emb], axis=0)

    ei_g = all_edge_index.astype(jnp.int32)
    sh_g = _shard_edges(ei_g[0], ei_g[1])
    sh_b = _shard_edges(edge_index_buy[0], edge_index_buy[1])
    sh_v = _shard_edges(edge_index_view[0], edge_index_view[1])

    ones_t = jnp.ones((N, D), jnp.float32)
    zer = jnp.zeros((ZS, D), jnp.float32)

    def agg_graph(table, shards):
        def one(s_, d_):
            nonlocal zer
            p = _sc_agg(table, s_, d_, zer).reshape(NC, NPAD, D)
            zer = lax.optimization_barrier((zer, p[0, :ZS]))[0]
            return p

        return jnp.concatenate([one(s_, d_) for (s_, d_) in shards], axis=0)

    degp = jnp.concatenate([
        agg_graph(ones_t, sh_g)[:, :, 0],
        agg_graph(ones_t, sh_b)[:, :, 0],
        agg_graph(ones_t, sh_v)[:, :, 0],
    ], axis=0)
    dis_all = _tc_dis(degp)
    dis_g = dis_all[0, :N][:, None]
    dis_buy = dis_all[1, :N][:, None]
    dis_view = dis_all[2, :N][:, None]

    def conv(x, w, b, dis, shards, base, scale):
        h = _tc_scale_mm(x, w, dis)
        p = agg_graph(h, shards)[:, :N, :]
        return _tc_post(p, dis, b, base, scale)

    x1, acc1 = conv(x0, Wg1, bg1, dis_g, sh_g, x0, 1.0)
    _, g = conv(x1, Wg2, bg2, dis_g, sh_g, acc1, 0.5)

    cascading = g
    outs = []
    for (w, b, uh, ih, dis, shards) in (
        (Wb_buy, bb_buy, uH_buy, iH_buy, dis_buy, sh_b),
        (Wb_view, bb_view, uH_view, iH_view, dis_view, sh_v),
    ):
        _, gcn_emb = conv(cascading, w, b, dis, shards, cascading, 1.0)
        sem = _tc_hyper(gcn_emb, uh, ih)
        cascading = _tc_cascade(cascading, gcn_emb, sem)
        outs.append(cascading)

    final_user = jnp.stack([o[:NU] for o in outs], axis=1)
    final_item = jnp.stack([o[NU:] for o in outs], axis=1)
    return (final_user, final_item)


# speedup vs baseline: 13.8877x; 1.3951x over previous
import functools

import jax
import jax.numpy as jnp
from jax import lax
from jax.experimental import pallas as pl
from jax.experimental.pallas import tpu as pltpu
from jax.experimental.pallas import tpu_sc as plsc

N_USERS = 4999
N_ITEMS = 4999
D = 128
N = N_USERS + 1 + N_ITEMS + 1
NU = N_USERS + 1

NC = 2
NS = 16
NW = NC * NS
CH = 128

NPAD = 10240
HS = NPAD + 128
ZCH = NPAD // NS
ZS = 40
NBUF = 2

_MESH = plsc.VectorSubcoreMesh(core_axis_name="c", subcore_axis_name="s")


CPW = 40


def _shard_edges(src, dst):
    e = src.shape[0]
    k = -(-e // (NW * CPW * CH))
    te = k * NW * CPW * CH
    pad = te - e
    ar = jnp.arange(pad, dtype=jnp.int32)
    src_p = jnp.concatenate([src.astype(jnp.int32), ar % 9973])
    dst_p = jnp.concatenate([dst.astype(jnp.int32), NPAD + (ar % 16)])
    src_p = src_p.reshape(k, NW, CPW, CH)
    dst_p = dst_p.reshape(k, NW, CPW, CH)
    return [(src_p[i], dst_p[i]) for i in range(k)]


def _sc_agg(table, src_sh, dst_sh, zeros_hbm):
    @functools.partial(
        pl.kernel,
        mesh=_MESH,
        out_type=jax.ShapeDtypeStruct((NC * NPAD, D), jnp.float32),
        scratch_types=[
            pltpu.VMEM((CPW, CH), jnp.int32),
            pltpu.VMEM((CPW, CH), jnp.int32),
            pltpu.VMEM((NBUF, CH, D), jnp.float32),
            pltpu.VMEM((ZS, D), jnp.float32),
            pltpu.VMEM_SHARED((HS, D), jnp.float32),
            pltpu.SemaphoreType.DMA,
        ],
    )
    def k(tab_h, src_h, dst_h, zer_h, out_h, src_v, dst_v, rows_v,
          stage_v, acc_sh, sem):
        c = lax.axis_index("c")
        s = lax.axis_index("s")
        wid = s * NC + c
        pltpu.sync_copy(src_h.at[wid], src_v)
        pltpu.sync_copy(dst_h.at[wid], dst_v)

        def fire(jj):
            pltpu.async_copy(
                tab_h.at[src_v.at[jj]], rows_v.at[lax.rem(jj, NBUF)], sem
            )

        pltpu.sync_copy(zer_h, stage_v)
        for zz in range(ZCH // ZS):
            pltpu.sync_copy(stage_v, acc_sh.at[pl.ds(s * ZCH + zz * ZS, ZS)])
        plsc.subcore_barrier()
        fire(0)

        def body(j, carry):
            @pl.when(j < CPW - 1)
            def _():
                fire(j + 1)

            buf = rows_v.at[lax.rem(j, NBUF)]
            pltpu.make_async_copy(tab_h.at[pl.ds(0, CH)], buf, sem).wait()
            pltpu.sync_copy(buf, acc_sh.at[dst_v.at[j]], add=True)
            return carry

        lax.fori_loop(0, CPW, body, 0)
        plsc.subcore_barrier()
        for zz in range(ZCH // ZS):
            pltpu.sync_copy(acc_sh.at[pl.ds(s * ZCH + zz * ZS, ZS)], stage_v)
            pltpu.sync_copy(
                stage_v,
                out_h.at[pl.ds(c * NPAD + s * ZCH + zz * ZS, ZS)],
            )
        plsc.subcore_barrier()

    return k(table, src_sh, dst_sh, zeros_hbm)


def _dis_body(p_ref, o_ref):
    p = p_ref[...]
    deg = jnp.stack([
        jnp.sum(p[0:4], axis=0),
        jnp.sum(p[4:6], axis=0),
        jnp.sum(p[6:8], axis=0),
    ])
    o_ref[...] = jnp.where(
        deg > 0, 1.0 / jnp.sqrt(jnp.maximum(deg, 1e-12)), 0.0
    )


def _tc_dis(degp):
    return pl.pallas_call(
        _dis_body,
        out_shape=jax.ShapeDtypeStruct((3, NPAD), jnp.float32),
    )(degp)


def _scale_mm_body(x_ref, w_ref, s_ref, o_ref):
    o_ref[...] = s_ref[...] * jnp.dot(
        x_ref[...], w_ref[...], preferred_element_type=jnp.float32
    )


def _tc_scale_mm(x, w, s):
    return pl.pallas_call(
        _scale_mm_body,
        out_shape=jax.ShapeDtypeStruct(x.shape, jnp.float32),
    )(x, w, s)


def _post_body(scale, p_ref, s_ref, b_ref, base_ref, y_ref, acc_ref):
    t = s_ref[...] * jnp.sum(p_ref[...], axis=0) + b_ref[...]
    nrm = jnp.sqrt(jnp.sum(t * t, axis=-1, keepdims=True))
    y = t / jnp.maximum(nrm, 1e-12)
    y_ref[...] = y
    acc_ref[...] = base_ref[...] + scale * y


def _tc_post(p, dis, b, base, scale):
    return pl.pallas_call(
        functools.partial(_post_body, scale),
        out_shape=[
            jax.ShapeDtypeStruct((N, D), jnp.float32),
            jax.ShapeDtypeStruct((N, D), jnp.float32),
        ],
    )(p, dis, b.reshape(1, D), base)


def _hyper_body(g_ref, uh_ref, ih_ref, o_ref):
    u = g_ref[0:NU, :]
    it = g_ref[NU:N, :]
    hu = jnp.dot(u, uh_ref[...], preferred_element_type=jnp.float32)
    hi = jnp.dot(it, ih_ref[...], preferred_element_type=jnp.float32)
    au = lax.dot_general(hu, u, (((0,), (0,)), ((), ())),
                         preferred_element_type=jnp.float32)
    ai = lax.dot_general(hi, it, (((0,), (0,)), ((), ())),
                         preferred_element_type=jnp.float32)
    o_ref[0:NU, :] = jnp.dot(hu, au, preferred_element_type=jnp.float32)
    o_ref[NU:N, :] = jnp.dot(hi, ai, preferred_element_type=jnp.float32)


def _tc_hyper(gcn, uh, ih):
    return pl.pallas_call(
        _hyper_body,
        out_shape=jax.ShapeDtypeStruct((N, D), jnp.float32),
    )(gcn, uh, ih)


def _cascade_body(temp_ref, col_ref, sem_ref, o_ref):
    col = col_ref[...]
    sem = sem_ref[...]
    num = jnp.sum(sem * col, axis=-1, keepdims=True)
    den = jnp.sum(col * col, axis=-1, keepdims=True) + 1e-08
    o_ref[...] = temp_ref[...] + col + (num / den) * col


def _tc_cascade(temp, gcn, sem):
    return pl.pallas_call(
        _cascade_body,
        out_shape=jax.ShapeDtypeStruct((N, D), jnp.float32),
    )(temp, gcn, sem)


def kernel(user_emb, item_emb, Wg1, bg1, Wg2, bg2, Wb_buy, bb_buy, Wb_view,
           bb_view, uH_buy, iH_buy, uH_view, iH_view, all_edge_index,
           edge_index_buy, edge_index_view):
    x0 = jnp.concatenate([user_emb, item_emb], axis=0)

    ei_g = all_edge_index.astype(jnp.int32)
    sh_g = _shard_edges(ei_g[0], ei_g[1])
    sh_b = _shard_edges(edge_index_buy[0], edge_index_buy[1])
    sh_v = _shard_edges(edge_index_view[0], edge_index_view[1])

    ones_t = jnp.ones((N, D), jnp.float32)
    zer = jnp.zeros((ZS, D), jnp.float32)

    def agg_graph(table, shards):
        def one(s_, d_):
            nonlocal zer
            p = _sc_agg(table, s_, d_, zer).reshape(NC, NPAD, D)
            zer = lax.optimization_barrier((zer, p[0, :ZS]))[0]
            return p

        return jnp.concatenate([one(s_, d_) for (s_, d_) in shards], axis=0)

    degp = jnp.concatenate([
        agg_graph(ones_t, sh_g)[:, :, 0],
        agg_graph(ones_t, sh_b)[:, :, 0],
        agg_graph(ones_t, sh_v)[:, :, 0],
    ], axis=0)
    dis_all = _tc_dis(degp)
    dis_g = dis_all[0, :N][:, None]
    dis_buy = dis_all[1, :N][:, None]
    dis_view = dis_all[2, :N][:, None]

    def conv(x, w, b, dis, shards, base, scale):
        h = _tc_scale_mm(x, w, dis)
        p = agg_graph(h, shards)[:, :N, :]
        return _tc_post(p, dis, b, base, scale)

    x1, acc1 = conv(x0, Wg1, bg1, dis_g, sh_g, x0, 1.0)
    _, g = conv(x1, Wg2, bg2, dis_g, sh_g, acc1, 0.5)

    cascading = g
    outs = []
    for (w, b, uh, ih, dis, shards) in (
        (Wb_buy, bb_buy, uH_buy, iH_buy, dis_buy, sh_b),
        (Wb_view, bb_view, uH_view, iH_view, dis_view, sh_v),
    ):
        _, gcn_emb = conv(cascading, w, b, dis, shards, cascading, 1.0)
        sem = _tc_hyper(gcn_emb, uh, ih)
        cascading = _tc_cascade(cascading, gcn_emb, sem)
        outs.append(cascading)

    final_user = jnp.stack([o[:NU] for o in outs], axis=1)
    final_item = jnp.stack([o[NU:] for o in outs], axis=1)
    return (final_user, final_item)
